# baseline (speedup 1.0000x reference)
import functools

import jax
import jax.numpy as jnp
from jax import lax
from jax.experimental import pallas as pl
from jax.experimental.pallas import tpu as pltpu
from jax.experimental.pallas import tpu_sc as plsc

L = 16
NC = 2
NS = 16
NW = NC * NS

EMBED = 16
HIDDEN = 8

BLOCK_W = 143360
OUT_ROWS = 7840


def _tc_scores(n: int):
    grid = (n + BLOCK_W - 1) // BLOCK_W

    def body(t_ref, w1_ref, b1_ref, w2_ref, o_ref):
        w1 = w1_ref[...]
        b1v = b1_ref[...]
        w2v = w2_ref[...]
        xs = t_ref[...]
        h = lax.dot_general(
            w1, xs, (((1,), (0,)), ((), ())),
            preferred_element_type=jnp.float32,
        )
        h = jnp.maximum(h + b1v, 0.0)
        sc = lax.dot_general(
            w2v, h, (((1,), (0,)), ((), ())),
            preferred_element_type=jnp.float32,
        )
        for kk in range(BLOCK_W // 128):
            o_ref[pl.ds(kk, 1), :] = sc[:, kk * 128:(kk + 1) * 128]

    return pl.pallas_call(
        body,
        grid=(grid,),
        in_specs=[
            pl.BlockSpec((EMBED, BLOCK_W), lambda b: (0, b)),
            pl.BlockSpec((HIDDEN, EMBED), lambda b: (0, 0)),
            pl.BlockSpec((HIDDEN, 1), lambda b: (0, 0)),
            pl.BlockSpec((1, HIDDEN), lambda b: (0, 0)),
        ],
        out_specs=pl.BlockSpec((BLOCK_W // 128, 128), lambda b: (b, 0)),
        out_shape=jax.ShapeDtypeStruct((OUT_ROWS, 128), jnp.float32),
    )


def _sc_gather(B: int):
    b_per_w = B // NW
    nblk = b_per_w // L
    nchunk = b_per_w // 128
    mesh = plsc.VectorSubcoreMesh(core_axis_name="c", subcore_axis_name="s")
    cp = pltpu.CompilerParams(
        needs_layout_passes=False, use_tc_tiling_on_sc=False
    )

    @functools.partial(
        pl.kernel,
        mesh=mesh,
        compiler_params=cp,
        out_type=jax.ShapeDtypeStruct((B,), jnp.float32),
        scratch_types=[
            pltpu.VMEM((b_per_w,), jnp.int32),
            *[pltpu.VMEM((128,), jnp.int32) for _ in range(4)],
            pltpu.VMEM((b_per_w, L), jnp.float32),
            pltpu.VMEM((1, L), jnp.float32),
            pltpu.VMEM((b_per_w,), jnp.float32),
            pltpu.SemaphoreType.DMA,
        ],
    )
    def k(ids_hbm, scores_hbm, b2_hbm, out_hbm, idx_v, q0, q1, q2, q3,
          rows_v, b2_v, score_v, sem):
        wid = lax.axis_index("s") * NC + lax.axis_index("c")
        base = wid * b_per_w
        pltpu.sync_copy(b2_hbm, b2_v)
        pltpu.sync_copy(ids_hbm.at[pl.ds(base, b_per_w)], idx_v)

        qs = [q0, q1, q2, q3]
        for j in range(nchunk):
            for i in range(128 // L):
                v = idx_v[pl.ds(j * 128 + i * L, L)]
                qs[j][pl.ds(i * L, L)] = lax.shift_right_logical(v, 4)
        copies = [
            pltpu.async_copy(
                scores_hbm.at[qs[j]],
                rows_v.at[pl.ds(j * 128, 128), :],
                sem,
            )
            for j in range(nchunk)
        ]
        for c in copies:
            c.wait()

        lanes = lax.iota(jnp.int32, L)

        @pl.loop(0, nblk)
        def _(i):
            row0 = i * L
            ridx = row0 + lanes
            ids = idx_v[pl.ds(row0, L)]
            sc = plsc.load_gather(rows_v, [ridx, ids & (L - 1)])
            score_v[pl.ds(row0, L)] = sc + b2_v[0]

        pltpu.sync_copy(score_v, out_hbm.at[pl.ds(base, b_per_w)])

    return k


def kernel(title_ids, table, W1, b1, W2, b2):
    B = title_ids.shape[0]
    n = table.shape[0]
    scores2d = _tc_scores(n)(
        table.T, W1.T, b1.reshape(HIDDEN, 1), W2.reshape(1, HIDDEN)
    )
    b2b = jnp.broadcast_to(b2.reshape(1, 1), (1, L)).astype(jnp.float32)
    scores16 = scores2d.reshape(OUT_ROWS * 8, L)
    out = _sc_gather(B)(title_ids.astype(jnp.int32), scores16, b2b)
    return out.reshape(B, 1)


# speedup vs baseline: 1.0037x; 1.0037x over previous
import functools

import jax
import jax.numpy as jnp
from jax import lax
from jax.experimental import pallas as pl
from jax.experimental.pallas import tpu as pltpu
from jax.experimental.pallas import tpu_sc as plsc

L = 16
NC = 2
NS = 16
NW = NC * NS

EMBED = 16
HIDDEN = 8

BLOCK_W = 125952
OUT_ROWS = 7872


def _tc_scores(n: int):
    grid = (n + BLOCK_W - 1) // BLOCK_W

    def body(t_ref, w1_ref, b1_ref, w2_ref, o_ref):
        w1 = w1_ref[...]
        b1v = b1_ref[...]
        w2v = w2_ref[...]
        xs = t_ref[...]
        h = lax.dot_general(
            w1, xs, (((1,), (0,)), ((), ())),
            preferred_element_type=jnp.float32,
        )
        h = jnp.maximum(h + b1v, 0.0)
        sc = lax.dot_general(
            w2v, h, (((1,), (0,)), ((), ())),
            preferred_element_type=jnp.float32,
        )
        for kk in range(BLOCK_W // 128):
            o_ref[pl.ds(kk, 1), :] = sc[:, kk * 128:(kk + 1) * 128]

    return pl.pallas_call(
        body,
        grid=(grid,),
        in_specs=[
            pl.BlockSpec((EMBED, BLOCK_W), lambda b: (0, b)),
            pl.BlockSpec((HIDDEN, EMBED), lambda b: (0, 0)),
            pl.BlockSpec((HIDDEN, 1), lambda b: (0, 0)),
            pl.BlockSpec((1, HIDDEN), lambda b: (0, 0)),
        ],
        out_specs=pl.BlockSpec((BLOCK_W // 128, 128), lambda b: (b, 0)),
        out_shape=jax.ShapeDtypeStruct((OUT_ROWS, 128), jnp.float32),
    )


def _sc_gather(B: int):
    b_per_w = B // NW
    nblk = b_per_w // L
    nchunk = b_per_w // 128
    mesh = plsc.VectorSubcoreMesh(core_axis_name="c", subcore_axis_name="s")
    cp = pltpu.CompilerParams(
        needs_layout_passes=False, use_tc_tiling_on_sc=False
    )

    @functools.partial(
        pl.kernel,
        mesh=mesh,
        compiler_params=cp,
        out_type=jax.ShapeDtypeStruct((B,), jnp.float32),
        scratch_types=[
            pltpu.VMEM((b_per_w,), jnp.int32),
            *[pltpu.VMEM((128,), jnp.int32) for _ in range(4)],
            pltpu.VMEM((b_per_w, L), jnp.float32),
            pltpu.VMEM((1, L), jnp.float32),
            pltpu.VMEM((b_per_w,), jnp.float32),
            pltpu.SemaphoreType.DMA,
        ],
    )
    def k(ids_hbm, scores_hbm, b2_hbm, out_hbm, idx_v, q0, q1, q2, q3,
          rows_v, b2_v, score_v, sem):
        wid = lax.axis_index("s") * NC + lax.axis_index("c")
        base = wid * b_per_w
        pltpu.sync_copy(b2_hbm, b2_v)
        pltpu.sync_copy(ids_hbm.at[pl.ds(base, b_per_w)], idx_v)

        qs = [q0, q1, q2, q3]
        for j in range(nchunk):
            for i in range(128 // L):
                v = idx_v[pl.ds(j * 128 + i * L, L)]
                qs[j][pl.ds(i * L, L)] = lax.shift_right_logical(v, 4)
        copies = [
            pltpu.async_copy(
                scores_hbm.at[qs[j]],
                rows_v.at[pl.ds(j * 128, 128), :],
                sem,
            )
            for j in range(nchunk)
        ]
        for c in copies:
            c.wait()

        lanes = lax.iota(jnp.int32, L)

        @pl.loop(0, nblk)
        def _(i):
            row0 = i * L
            ridx = row0 + lanes
            ids = idx_v[pl.ds(row0, L)]
            sc = plsc.load_gather(rows_v, [ridx, ids & (L - 1)])
            score_v[pl.ds(row0, L)] = sc + b2_v[0]

        pltpu.sync_copy(score_v, out_hbm.at[pl.ds(base, b_per_w)])

    return k


def kernel(title_ids, table, W1, b1, W2, b2):
    B = title_ids.shape[0]
    n = table.shape[0]
    scores2d = _tc_scores(n)(
        table.T, W1.T, b1.reshape(HIDDEN, 1), W2.reshape(1, HIDDEN)
    )
    b2b = jnp.broadcast_to(b2.reshape(1, 1), (1, L)).astype(jnp.float32)
    scores16 = scores2d.reshape(OUT_ROWS * 8, L)
    out = _sc_gather(B)(title_ids.astype(jnp.int32), scores16, b2b)
    return out.reshape(B, 1)
